# Initial kernel scaffold; baseline (speedup 1.0000x reference)
import jax
import jax.numpy as jnp
from jax.experimental import pallas as pl


def kernel(z, embedding_weight):
    raise NotImplementedError("write your pallas kernel here")


# speedup vs baseline: 1.2594x; 1.2594x over previous
import jax
import jax.numpy as jnp
from jax.experimental import pallas as pl
from jax.experimental.pallas import tpu as pltpu
from jax.experimental.pallas import tpu_sc as plsc

_N_E = 8192
_D = 256
_N_TOK = 4608
_BM = 512
_BN = 1024
_NI = _N_TOK // _BM
_NJ = _N_E // _BN
_GW = 128
_EPS = 1e-12


def _argmin_body(z_ref, e_ref, idx_ref, loss_ref,
                 enb_s, esq_s, znm2_s, zsq_s, rv_s, ri_s):
    i = pl.program_id(0)
    j = pl.program_id(1)

    @pl.when(i == 0)
    def _prep_codebook():
        e = e_ref[...]
        nrm = jnp.sqrt(jnp.sum(e * e, axis=1, keepdims=True))
        en = e / jnp.maximum(nrm, _EPS)
        enb_s[j] = en.astype(jnp.bfloat16)
        esq = jnp.sum(en * en, axis=1, keepdims=True)
        esq_s[j] = esq.reshape(1, _BN)

    @pl.when(j == 0)
    def _prep_z():
        zb = z_ref[...]
        nrm = jnp.sqrt(jnp.sum(zb * zb, axis=1, keepdims=True))
        zn = zb / jnp.maximum(nrm, _EPS)
        zsq_s[...] = jnp.sum(zn * zn, keepdims=True)
        znm2_s[...] = (-2.0 * zn).astype(jnp.bfloat16)
        rv_s[...] = jnp.full((_BM, 128), jnp.inf, jnp.float32)
        ri_s[...] = jnp.zeros((_BM, 128), jnp.int32)

    d2 = jax.lax.dot_general(
        znm2_s[...], enb_s[j],
        dimension_numbers=(((1,), (1,)), ((), ())),
        preferred_element_type=jnp.float32)
    score = d2 + esq_s[j]

    lane = jax.lax.broadcasted_iota(jnp.int32, (_BM, 128), 1)
    rv = rv_s[...]
    ri = ri_s[...]
    for g in range(_BN // 128):
        sg = score[:, g * 128:(g + 1) * 128]
        ig = lane + (j * _BN + g * 128)
        m = sg < rv
        rv = jnp.where(m, sg, rv)
        ri = jnp.where(m, ig, ri)
    rv_s[...] = rv
    ri_s[...] = ri

    @pl.when(j == _NJ - 1)
    def _finalize():
        mv = jnp.min(rv, axis=1, keepdims=True)
        cand = jnp.where(rv == mv, ri, jnp.full((_BM, 128), 2**31 - 1,
                                                jnp.int32))
        idx_ref[...] = jnp.min(cand, axis=1, keepdims=True)
        part = zsq_s[...] + jnp.sum(mv, keepdims=True)
        prev = jnp.where(i == 0, jnp.zeros((1, 1), jnp.float32),
                         loss_ref[...])
        tot = prev + part
        scale = 1.25 / float(_N_TOK * _D)
        tot = jnp.where(i == _NI - 1, tot * scale, tot)
        loss_ref[...] = tot


def _argmin_call(z_flat, emb):
    return pl.pallas_call(
        _argmin_body,
        grid=(_NI, _NJ),
        in_specs=[
            pl.BlockSpec((_BM, _D), lambda i, j: (i, 0)),
            pl.BlockSpec((_BN, _D),
                         lambda i, j: (jnp.where(i == 0, j, _NJ - 1), 0)),
        ],
        out_specs=[
            pl.BlockSpec((_BM, 1), lambda i, j: (i, 0)),
            pl.BlockSpec((1, 1), lambda i, j: (0, 0)),
        ],
        out_shape=[
            jax.ShapeDtypeStruct((_N_TOK, 1), jnp.int32),
            jax.ShapeDtypeStruct((1, 1), jnp.float32),
        ],
        scratch_shapes=[
            pltpu.VMEM((_NJ, _BN, _D), jnp.bfloat16),
            pltpu.VMEM((_NJ, 1, _BN), jnp.float32),
            pltpu.VMEM((_BM, _D), jnp.bfloat16),
            pltpu.VMEM((1, 1), jnp.float32),
            pltpu.VMEM((_BM, 128), jnp.float32),
            pltpu.VMEM((_BM, 128), jnp.int32),
        ],
        compiler_params=pltpu.CompilerParams(
            dimension_semantics=("arbitrary", "arbitrary")),
    )(z_flat, emb)


def _gather_rows(table, idx_row):
    mesh = plsc.VectorSubcoreMesh(core_axis_name="core",
                                  subcore_axis_name="subcore")

    @pl.kernel(out_type=jax.ShapeDtypeStruct((_N_TOK, _D), table.dtype),
               mesh=mesh)
    def k(x_hbm, i_hbm, o_hbm):
        def body(i_vmem, o_vmem):
            pltpu.sync_copy(x_hbm.at[i_vmem.at[0]], o_vmem)

        pltpu.emit_pipeline(
            body,
            grid=(_N_TOK // _GW,),
            in_specs=[pl.BlockSpec((1, _GW), index_map=lambda i: (0, i))],
            out_specs=[pl.BlockSpec((_GW, _D), index_map=lambda i: (i, 0))],
            core_axis_name="subcore",
            dimension_semantics=(pltpu.PARALLEL,),
        )(i_hbm, o_hbm)

    return k(table, idx_row)


def _normalize_body(x_ref, o_ref):
    x = x_ref[...]
    nrm = jnp.sqrt(jnp.sum(x * x, axis=1, keepdims=True))
    o_ref[...] = x / jnp.maximum(nrm, _EPS)


def _normalize_call(x):
    nb = 4
    bm = _N_TOK // nb
    return pl.pallas_call(
        _normalize_body,
        grid=(nb,),
        in_specs=[pl.BlockSpec((bm, _D), lambda i: (i, 0))],
        out_specs=pl.BlockSpec((bm, _D), lambda i: (i, 0)),
        out_shape=jax.ShapeDtypeStruct((_N_TOK, _D), jnp.float32),
    )(x)


def kernel(z, embedding_weight):
    z_flat = z.reshape(_N_TOK, _D)
    idx2, loss = _argmin_call(z_flat, embedding_weight)
    idx = idx2.reshape(_N_TOK)
    z_q = _gather_rows(embedding_weight, idx2.reshape(1, _N_TOK))
    z_qnorm = _normalize_call(z_q)
    return (z_qnorm.reshape(z.shape), loss.reshape(()), idx)


# speedup vs baseline: 1.6907x; 1.3425x over previous
import jax
import jax.numpy as jnp
from jax.experimental import pallas as pl
from jax.experimental.pallas import tpu as pltpu
from jax.experimental.pallas import tpu_sc as plsc

_N_E = 8192
_D = 256
_N_TOK = 4608
_BM = 512
_BN = 1024
_NI = _N_TOK // _BM
_NJ = _N_E // _BN
_GW = 128
_EPS = 1e-12


def _argmin_body(z_ref, e_ref, idx_ref, loss_ref, enb_s, esq_s):
    i = pl.program_id(0)

    @pl.when(i == 0)
    def _prep_codebook():
        for j in range(_NJ):
            e = e_ref[j * _BN:(j + 1) * _BN, :]
            nrm = jnp.sqrt(jnp.sum(e * e, axis=1, keepdims=True))
            en = e / jnp.maximum(nrm, _EPS)
            enb_s[j] = en.astype(jnp.bfloat16)
            esq = jnp.sum(en * en, axis=1, keepdims=True)
            esq_s[j] = esq.reshape(1, _BN)

    zb = z_ref[...]
    nrm = jnp.sqrt(jnp.sum(zb * zb, axis=1, keepdims=True))
    zn = zb / jnp.maximum(nrm, _EPS)
    zsq = jnp.sum(zn * zn, keepdims=True)
    znm2 = (-2.0 * zn).astype(jnp.bfloat16)

    nr = _BM // 128
    rv = [None] * nr
    ri = [None] * nr
    for j in range(_NJ):
        d2 = jax.lax.dot_general(
            znm2, enb_s[j],
            dimension_numbers=(((1,), (1,)), ((), ())),
            preferred_element_type=jnp.float32)
        score = d2 + esq_s[j]
        for r in range(nr):
            rvc, ric = rv[r], ri[r]
            for g in range(_BN // 128):
                sg = score[r * 128:(r + 1) * 128, g * 128:(g + 1) * 128]
                fold = j * (_BN // 128) + g
                if rvc is None:
                    rvc = sg
                    ric = jnp.zeros((128, 128), jnp.int32)
                else:
                    m = sg < rvc
                    rvc = jnp.where(m, sg, rvc)
                    ric = jnp.where(m, jnp.full((128, 128), fold,
                                                jnp.int32), ric)
            rv[r], ri[r] = rvc, ric

    rva = jnp.concatenate(rv, axis=0)
    ria = jnp.concatenate(ri, axis=0)
    lane = jax.lax.broadcasted_iota(jnp.int32, (_BM, 128), 1)
    gidx = ria * 128 + lane
    mv = jnp.min(rva, axis=1, keepdims=True)
    cand = jnp.where(rva == mv, gidx, jnp.full((_BM, 128), 2**31 - 1,
                                               jnp.int32))
    idx_ref[...] = jnp.min(cand, axis=1, keepdims=True)
    part = zsq + jnp.sum(mv, keepdims=True)
    prev = jnp.where(i == 0, jnp.zeros((1, 1), jnp.float32),
                     loss_ref[...])
    tot = prev + part
    scale = 1.25 / float(_N_TOK * _D)
    tot = jnp.where(i == _NI - 1, tot * scale, tot)
    loss_ref[...] = tot


def _argmin_call(z_flat, emb):
    return pl.pallas_call(
        _argmin_body,
        grid=(_NI,),
        in_specs=[
            pl.BlockSpec((_BM, _D), lambda i: (i, 0)),
            pl.BlockSpec((_N_E, _D), lambda i: (0, 0)),
        ],
        out_specs=[
            pl.BlockSpec((_BM, 1), lambda i: (i, 0)),
            pl.BlockSpec((1, 1), lambda i: (0, 0)),
        ],
        out_shape=[
            jax.ShapeDtypeStruct((_N_TOK, 1), jnp.int32),
            jax.ShapeDtypeStruct((1, 1), jnp.float32),
        ],
        scratch_shapes=[
            pltpu.VMEM((_NJ, _BN, _D), jnp.bfloat16),
            pltpu.VMEM((_NJ, 1, _BN), jnp.float32),
        ],
        compiler_params=pltpu.CompilerParams(
            dimension_semantics=("arbitrary",)),
    )(z_flat, emb)


def _gather_rows(table, idx_row):
    mesh = plsc.VectorSubcoreMesh(core_axis_name="core",
                                  subcore_axis_name="subcore")

    @pl.kernel(out_type=jax.ShapeDtypeStruct((_N_TOK, _D), table.dtype),
               mesh=mesh)
    def k(x_hbm, i_hbm, o_hbm):
        def body(i_vmem, o_vmem):
            pltpu.sync_copy(x_hbm.at[i_vmem.at[0]], o_vmem)

        pltpu.emit_pipeline(
            body,
            grid=(_N_TOK // _GW,),
            in_specs=[pl.BlockSpec((1, _GW), index_map=lambda i: (0, i))],
            out_specs=[pl.BlockSpec((_GW, _D), index_map=lambda i: (i, 0))],
            core_axis_name="subcore",
            dimension_semantics=(pltpu.PARALLEL,),
        )(i_hbm, o_hbm)

    return k(table, idx_row)


def _normalize_body(x_ref, o_ref):
    x = x_ref[...]
    nrm = jnp.sqrt(jnp.sum(x * x, axis=1, keepdims=True))
    o_ref[...] = x / jnp.maximum(nrm, _EPS)


def _normalize_call(x):
    nb = 4
    bm = _N_TOK // nb
    return pl.pallas_call(
        _normalize_body,
        grid=(nb,),
        in_specs=[pl.BlockSpec((bm, _D), lambda i: (i, 0))],
        out_specs=pl.BlockSpec((bm, _D), lambda i: (i, 0)),
        out_shape=jax.ShapeDtypeStruct((_N_TOK, _D), jnp.float32),
    )(x)


def kernel(z, embedding_weight):
    z_flat = z.reshape(_N_TOK, _D)
    idx2, loss = _argmin_call(z_flat, embedding_weight)
    idx = idx2.reshape(_N_TOK)
    z_q = _gather_rows(embedding_weight, idx2.reshape(1, _N_TOK))
    z_qnorm = _normalize_call(z_q)
    return (z_qnorm.reshape(z.shape), loss.reshape(()), idx)


# speedup vs baseline: 1.6969x; 1.0037x over previous
import jax
import jax.numpy as jnp
from jax.experimental import pallas as pl
from jax.experimental.pallas import tpu as pltpu
from jax.experimental.pallas import tpu_sc as plsc

_N_E = 8192
_D = 256
_N_TOK = 4608
_BM = 512
_BN = 1024
_NI = _N_TOK // _BM
_NJ = _N_E // _BN
_GW = 128
_EPS = 1e-12


def _argmin_body(z_ref, e_ref, idx_ref, loss_ref, enb_s, esq_s):
    i = pl.program_id(0)

    @pl.when(i == 0)
    def _prep_codebook():
        for j in range(_NJ):
            e = e_ref[j * _BN:(j + 1) * _BN, :]
            nrm = jnp.sqrt(jnp.sum(e * e, axis=1, keepdims=True))
            en = e / jnp.maximum(nrm, _EPS)
            enb_s[j] = en.astype(jnp.bfloat16)
            esq = jnp.sum(en * en, axis=1, keepdims=True)
            esq_s[j] = esq.reshape(1, _BN)

    zb = z_ref[...]
    nrm = jnp.sqrt(jnp.sum(zb * zb, axis=1, keepdims=True))
    zn = zb / jnp.maximum(nrm, _EPS)
    zsq = jnp.sum(zn * zn, keepdims=True)
    znm2 = (-2.0 * zn).astype(jnp.bfloat16)

    nr = _BM // 128
    rv = [None] * nr
    ri = [None] * nr
    for j in range(_NJ):
        d2 = jax.lax.dot_general(
            znm2, enb_s[j],
            dimension_numbers=(((1,), (1,)), ((), ())),
            preferred_element_type=jnp.float32)
        score = d2 + esq_s[j]
        for r in range(nr):
            rvc, ric = rv[r], ri[r]
            for g in range(_BN // 128):
                sg = score[r * 128:(r + 1) * 128, g * 128:(g + 1) * 128]
                fold = j * (_BN // 128) + g
                if rvc is None:
                    rvc = sg
                    ric = jnp.zeros((128, 128), jnp.int32)
                else:
                    m = sg < rvc
                    rvc = jnp.where(m, sg, rvc)
                    ric = jnp.where(m, jnp.full((128, 128), fold,
                                                jnp.int32), ric)
            rv[r], ri[r] = rvc, ric

    rva = jnp.concatenate(rv, axis=0)
    ria = jnp.concatenate(ri, axis=0)
    lane = jax.lax.broadcasted_iota(jnp.int32, (_BM, 128), 1)
    gidx = ria * 128 + lane
    mv = jnp.min(rva, axis=1, keepdims=True)
    cand = jnp.where(rva == mv, gidx, jnp.full((_BM, 128), 2**31 - 1,
                                               jnp.int32))
    idx_ref[...] = jnp.min(cand, axis=1, keepdims=True)
    part = zsq + jnp.sum(mv, keepdims=True)
    prev = jnp.where(i == 0, jnp.zeros((1, 1), jnp.float32),
                     loss_ref[...])
    tot = prev + part
    scale = 1.25 / float(_N_TOK * _D)
    tot = jnp.where(i == _NI - 1, tot * scale, tot)
    loss_ref[...] = tot


def _argmin_call(z_flat, emb):
    return pl.pallas_call(
        _argmin_body,
        grid=(_NI,),
        in_specs=[
            pl.BlockSpec((_BM, _D), lambda i: (i, 0)),
            pl.BlockSpec((_N_E, _D), lambda i: (0, 0)),
        ],
        out_specs=[
            pl.BlockSpec((_BM, 1), lambda i: (i, 0)),
            pl.BlockSpec((1, 1), lambda i: (0, 0)),
        ],
        out_shape=[
            jax.ShapeDtypeStruct((_N_TOK, 1), jnp.int32),
            jax.ShapeDtypeStruct((1, 1), jnp.float32),
        ],
        scratch_shapes=[
            pltpu.VMEM((_NJ, _BN, _D), jnp.bfloat16),
            pltpu.VMEM((_NJ, 1, _BN), jnp.float32),
        ],
        compiler_params=pltpu.CompilerParams(
            dimension_semantics=("arbitrary",)),
    )(z_flat, emb)


def _gather_rows(table, idx_row):
    mesh = plsc.VectorSubcoreMesh(core_axis_name="core",
                                  subcore_axis_name="subcore")

    @pl.kernel(out_type=jax.ShapeDtypeStruct((_N_TOK, _D), table.dtype),
               mesh=mesh)
    def k(x_hbm, i_hbm, o_hbm):
        def body(i_vmem, o_vmem):
            pltpu.sync_copy(x_hbm.at[i_vmem.at[0]], o_vmem)

        pltpu.emit_pipeline(
            body,
            grid=(_N_TOK // _GW,),
            in_specs=[pl.BlockSpec((1, _GW), index_map=lambda i: (0, i))],
            out_specs=[pl.BlockSpec((_GW, _D), index_map=lambda i: (i, 0))],
            core_axis_name=("core", "subcore"),
            dimension_semantics=(pltpu.PARALLEL,),
        )(i_hbm, o_hbm)

    return k(table, idx_row)


def _normalize_body(x_ref, o_ref):
    x = x_ref[...]
    nrm = jnp.sqrt(jnp.sum(x * x, axis=1, keepdims=True))
    o_ref[...] = x / jnp.maximum(nrm, _EPS)


def _normalize_call(x):
    nb = 12
    bm = _N_TOK // nb
    return pl.pallas_call(
        _normalize_body,
        grid=(nb,),
        in_specs=[pl.BlockSpec((bm, _D), lambda i: (i, 0))],
        out_specs=pl.BlockSpec((bm, _D), lambda i: (i, 0)),
        out_shape=jax.ShapeDtypeStruct((_N_TOK, _D), jnp.float32),
    )(x)


def kernel(z, embedding_weight):
    z_flat = z.reshape(_N_TOK, _D)
    idx2, loss = _argmin_call(z_flat, embedding_weight)
    idx = idx2.reshape(_N_TOK)
    z_q = _gather_rows(embedding_weight, idx2.reshape(1, _N_TOK))
    z_qnorm = _normalize_call(z_q)
    return (z_qnorm.reshape(z.shape), loss.reshape(()), idx)


# speedup vs baseline: 1.9141x; 1.1280x over previous
import jax
import jax.numpy as jnp
from jax.experimental import pallas as pl
from jax.experimental.pallas import tpu as pltpu
from jax.experimental.pallas import tpu_sc as plsc

_N_E = 8192
_D = 256
_N_TOK = 4608
_BM = 512
_BN = 1024
_NI = _N_TOK // _BM
_NJ = _N_E // _BN
_GW = 128
_EPS = 1e-12


def _argmin_body(z_ref, e_ref, idx_ref, loss_ref, enb_s, esq_s):
    i = pl.program_id(0)

    @pl.when(i == 0)
    def _prep_codebook():
        for j in range(_NJ):
            e = e_ref[j * _BN:(j + 1) * _BN, :]
            nrm = jnp.sqrt(jnp.sum(e * e, axis=1, keepdims=True))
            en = e / jnp.maximum(nrm, _EPS)
            enb_s[j] = en.astype(jnp.bfloat16)
            esq = jnp.sum(en * en, axis=1, keepdims=True)
            esq_s[j] = esq.reshape(1, _BN)

    zb = z_ref[...]
    nrm = jnp.sqrt(jnp.sum(zb * zb, axis=1, keepdims=True))
    zn = zb / jnp.maximum(nrm, _EPS)
    zsq = jnp.sum(zn * zn, keepdims=True)
    znm2 = (-2.0 * zn).astype(jnp.bfloat16)

    nr = _BM // 128
    rv = [None] * nr
    ri = [None] * nr
    for j in range(_NJ):
        d2 = jax.lax.dot_general(
            znm2, enb_s[j],
            dimension_numbers=(((1,), (1,)), ((), ())),
            preferred_element_type=jnp.float32)
        score = d2 + esq_s[j]
        for r in range(nr):
            rvc, ric = rv[r], ri[r]
            for g in range(_BN // 128):
                sg = score[r * 128:(r + 1) * 128, g * 128:(g + 1) * 128]
                fold = j * (_BN // 128) + g
                if rvc is None:
                    rvc = sg
                    ric = jnp.zeros((128, 128), jnp.int32)
                else:
                    m = sg < rvc
                    rvc = jnp.where(m, sg, rvc)
                    ric = jnp.where(m, jnp.full((128, 128), fold,
                                                jnp.int32), ric)
            rv[r], ri[r] = rvc, ric

    rva = jnp.concatenate(rv, axis=0)
    ria = jnp.concatenate(ri, axis=0)
    rvt = rva.T
    rit = ria.T
    sub = jax.lax.broadcasted_iota(jnp.int32, (128, _BM), 0)
    gidx = rit * 128 + sub
    mv = jnp.min(rvt, axis=0, keepdims=True)
    cand = jnp.where(rvt == mv, gidx, jnp.full((128, _BM), 2**31 - 1,
                                               jnp.int32))
    idx_ref[...] = jnp.min(cand, axis=0, keepdims=True).reshape(1, 1, _BM)
    part = zsq + jnp.sum(mv, keepdims=True)
    prev = jnp.where(i == 0, jnp.zeros((1, 1), jnp.float32),
                     loss_ref[...])
    tot = prev + part
    scale = 1.25 / float(_N_TOK * _D)
    tot = jnp.where(i == _NI - 1, tot * scale, tot)
    loss_ref[...] = tot


def _argmin_call(z_flat, emb):
    return pl.pallas_call(
        _argmin_body,
        grid=(_NI,),
        in_specs=[
            pl.BlockSpec((_BM, _D), lambda i: (i, 0)),
            pl.BlockSpec((_N_E, _D), lambda i: (0, 0)),
        ],
        out_specs=[
            pl.BlockSpec((1, 1, _BM), lambda i: (i, 0, 0)),
            pl.BlockSpec((1, 1), lambda i: (0, 0)),
        ],
        out_shape=[
            jax.ShapeDtypeStruct((_NI, 1, _BM), jnp.int32),
            jax.ShapeDtypeStruct((1, 1), jnp.float32),
        ],
        scratch_shapes=[
            pltpu.VMEM((_NJ, _BN, _D), jnp.bfloat16),
            pltpu.VMEM((_NJ, 1, _BN), jnp.float32),
        ],
        compiler_params=pltpu.CompilerParams(
            dimension_semantics=("arbitrary",)),
    )(z_flat, emb)


def _gather_rows(table, idx_row):
    mesh = plsc.VectorSubcoreMesh(core_axis_name="core",
                                  subcore_axis_name="subcore")

    @pl.kernel(out_type=jax.ShapeDtypeStruct((_N_TOK, _D), table.dtype),
               mesh=mesh)
    def k(x_hbm, i_hbm, o_hbm):
        def body(i_vmem, o_vmem):
            pltpu.sync_copy(x_hbm.at[i_vmem.at[0]], o_vmem)

        pltpu.emit_pipeline(
            body,
            grid=(_N_TOK // _GW,),
            in_specs=[pl.BlockSpec((1, _GW), index_map=lambda i: (0, i))],
            out_specs=[pl.BlockSpec((_GW, _D), index_map=lambda i: (i, 0))],
            core_axis_name=("core", "subcore"),
            dimension_semantics=(pltpu.PARALLEL,),
        )(i_hbm, o_hbm)

    return k(table, idx_row)


def _normalize_body(x_ref, o_ref):
    x = x_ref[...]
    nrm = jnp.sqrt(jnp.sum(x * x, axis=1, keepdims=True))
    o_ref[...] = x / jnp.maximum(nrm, _EPS)


def _normalize_call(x):
    nb = 4
    bm = _N_TOK // nb
    return pl.pallas_call(
        _normalize_body,
        grid=(nb,),
        in_specs=[pl.BlockSpec((bm, _D), lambda i: (i, 0))],
        out_specs=pl.BlockSpec((bm, _D), lambda i: (i, 0)),
        out_shape=jax.ShapeDtypeStruct((_N_TOK, _D), jnp.float32),
    )(x)


def kernel(z, embedding_weight):
    z_flat = z.reshape(_N_TOK, _D)
    idx2, loss = _argmin_call(z_flat, embedding_weight)
    idx = idx2.reshape(_N_TOK)
    z_q = _gather_rows(embedding_weight, idx2.reshape(1, _N_TOK))
    z_qnorm = _normalize_call(z_q)
    return (z_qnorm.reshape(z.shape), loss.reshape(()), idx)
